# baseline (speedup 1.0000x reference)
import jax
import jax.numpy as jnp
from jax import lax
from jax.experimental import pallas as pl
from jax.experimental.pallas import tpu as pltpu
from jax.experimental.pallas import tpu_sc as plsc

NUM_EXPERTS = 64
TOP_K = 8
N_GROUP = 8
GROUP_SIZE = 8
TOPK_GROUP = 4
N_CAND = TOPK_GROUP * GROUP_SIZE
ROUTED_SCALING_FACTOR = 2.5

BT = 1024
NC, NS, L = 2, 16, 16
NW = NC * NS


def _score_block(h_ref, wt_ref, o_ref):
    logits = jnp.dot(h_ref[...], wt_ref[...], preferred_element_type=jnp.float32)
    o_ref[...] = jax.nn.sigmoid(logits.T)


def _scores_tc(hs, wt):
    n, hidden = hs.shape
    return pl.pallas_call(
        _score_block,
        grid=(n // BT,),
        in_specs=[
            pl.BlockSpec((BT, hidden), lambda i: (i, 0)),
            pl.BlockSpec((hidden, NUM_EXPERTS), lambda i: (0, 0)),
        ],
        out_specs=pl.BlockSpec((NUM_EXPERTS, BT), lambda i: (0, i)),
        out_shape=jax.ShapeDtypeStruct((NUM_EXPERTS, n), jnp.float32),
        compiler_params=pltpu.CompilerParams(
            dimension_semantics=("arbitrary",),
        ),
    )(hs, wt)


def _full(v, dtype=jnp.int32):
    return jnp.full((L,), v, dtype)


def _tourney(pairs):
    while len(pairs) > 1:
        nxt = []
        for a, b in zip(pairs[0::2], pairs[1::2]):
            take = (b[0] > a[0]) | ((b[0] == a[0]) & (b[1] < a[1]))
            nxt.append((jnp.where(take, b[0], a[0]),
                        jnp.where(take, b[1], a[1])))
        pairs = nxt
    return pairs[0]


def _route_body(s_hbm, idx_hbm, w_hbm, s_v, idx_v, w_v):
    spt = s_v.shape[1]
    wid = lax.axis_index("s") * NC + lax.axis_index("c")
    base = wid * spt
    pltpu.sync_copy(s_hbm.at[:, pl.ds(base, spt)], s_v)

    def step(t, carry):
        tok = t * L

        def col(e):
            return s_v[e, pl.ds(tok, L)]

        gs = []
        for g in range(N_GROUP):
            m1 = col(GROUP_SIZE * g)
            m2 = _full(-1e30, jnp.float32)
            for e in range(1, GROUP_SIZE):
                x = col(GROUP_SIZE * g + e)
                m2 = jnp.maximum(m2, jnp.minimum(m1, x))
                m1 = jnp.maximum(m1, x)
            gs.append(m1 + m2)

        ranks = []
        for g in range(N_GROUP):
            cnt = _full(0)
            for o in range(N_GROUP):
                if o == g:
                    continue
                m = (gs[o] >= gs[g]) if o < g else (gs[o] > gs[g])
                cnt = cnt + jnp.where(m, _full(1), _full(0))
            ranks.append(cnt)
        gsel = []
        for j in range(TOPK_GROUP):
            acc = _full(0)
            for g in range(N_GROUP):
                acc = jnp.where(ranks[g] == j, g, acc)
            gsel.append(acc)

        cand = [None] * N_CAND
        for i in range(GROUP_SIZE):
            cols = [col(GROUP_SIZE * g + i) for g in range(N_GROUP)]
            for j in range(TOPK_GROUP):
                acc = cols[0]
                for g in range(1, N_GROUP):
                    acc = jnp.where(ranks[g] == j, cols[g], acc)
                cand[GROUP_SIZE * j + i] = acc

        eids = []
        for j in range(TOPK_GROUP):
            gbase = gsel[j] * GROUP_SIZE
            for i in range(GROUP_SIZE):
                eids.append(gbase + i)

        wk = []
        wsum = _full(0.0, jnp.float32)
        for k in range(TOP_K):
            pairs = [(cand[e], eids[e]) for e in range(N_CAND)]
            m, win_eid = _tourney(pairs)
            for e in range(N_CAND):
                cand[e] = jnp.where(win_eid == eids[e],
                                    _full(-1e30, jnp.float32), cand[e])
            idx_v[pl.ds(t * (TOP_K * L) + k * L, L)] = win_eid
            wk.append(m)
            wsum = wsum + m
        scale = ROUTED_SCALING_FACTOR / (wsum + 1e-20)
        for k in range(TOP_K):
            w_v[pl.ds(t * (TOP_K * L) + k * L, L)] = wk[k] * scale
        return carry

    lax.fori_loop(0, spt // L, step, 0)
    pltpu.sync_copy(idx_v, idx_hbm.at[pl.ds(base * TOP_K, spt * TOP_K)])
    pltpu.sync_copy(w_v, w_hbm.at[pl.ds(base * TOP_K, spt * TOP_K)])


def _route_sc(scores_t):
    n = scores_t.shape[1]
    spt = n // NW
    mesh = plsc.VectorSubcoreMesh(core_axis_name="c", subcore_axis_name="s")
    fn = pl.kernel(
        _route_body,
        out_type=[
            jax.ShapeDtypeStruct((n * TOP_K,), jnp.int32),
            jax.ShapeDtypeStruct((n * TOP_K,), jnp.float32),
        ],
        mesh=mesh,
        scratch_types=[
            pltpu.VMEM((NUM_EXPERTS, spt), jnp.float32),
            pltpu.VMEM((spt * TOP_K,), jnp.int32),
            pltpu.VMEM((spt * TOP_K,), jnp.float32),
        ],
    )
    return fn(scores_t)


def _unswizzle(flat, n):
    return flat.reshape(n // L, TOP_K, L).transpose(0, 2, 1).reshape(n, TOP_K)


N_CHUNK = 4


def kernel(hidden_states, weight, e_score_correction_bias):
    hidden = hidden_states.shape[-1]
    hs = hidden_states.reshape(-1, hidden)
    n = hs.shape[0]
    wt = weight.T
    cn = n // N_CHUNK
    idxs, ws = [], []
    for c in range(N_CHUNK):
        scores_t = _scores_tc(hs[c * cn:(c + 1) * cn], wt)
        idx_flat, w_flat = _route_sc(scores_t)
        idxs.append(_unswizzle(idx_flat, cn))
        ws.append(_unswizzle(w_flat, cn))
    return jnp.concatenate(idxs, axis=0), jnp.concatenate(ws, axis=0)


# speedup vs baseline: 2.8887x; 2.8887x over previous
import functools

import jax
import jax.numpy as jnp
from jax import lax
from jax.experimental import pallas as pl
from jax.experimental.pallas import tpu as pltpu

NUM_EXPERTS = 64
TOP_K = 8
N_GROUP = 8
GROUP_SIZE = NUM_EXPERTS // N_GROUP
TOPK_GROUP = 4
ROUTED_SCALING_FACTOR = 2.5

BT = 1024


def _gate_block(h1_ref, h2_ref, wt1_ref, wt2_ref, bias_ref, idx_ref, wgt_ref):
    logits = (jnp.dot(h1_ref[...], wt1_ref[...], preferred_element_type=jnp.float32)
              + jnp.dot(h2_ref[...], wt2_ref[...], preferred_element_type=jnp.float32))
    lt = logits.T
    scores = jax.nn.sigmoid(lt)
    sfc = scores + bias_ref[...]

    bt = lt.shape[1]
    rows8 = lax.broadcasted_iota(jnp.int32, (N_GROUP, bt), 0)

    gs_list = []
    for g in range(N_GROUP):
        sub = sfc[g * GROUP_SIZE:(g + 1) * GROUP_SIZE, :]
        m1 = jnp.max(sub, axis=0, keepdims=True)
        first = jnp.min(jnp.where(sub == m1, rows8, N_GROUP), axis=0, keepdims=True)
        m2 = jnp.max(jnp.where(rows8 == first, -jnp.inf, sub), axis=0, keepdims=True)
        gs_list.append(m1 + m2)
    gs = jnp.concatenate(gs_list, axis=0)

    masked_parts = []
    for g in range(N_GROUP):
        row = gs[g:g + 1, :]
        better = (gs > row) | ((gs == row) & (rows8 < g))
        rank = jnp.sum(better.astype(jnp.int32), axis=0, keepdims=True)
        keep = rank < TOPK_GROUP
        sub = sfc[g * GROUP_SIZE:(g + 1) * GROUP_SIZE, :]
        masked_parts.append(jnp.where(keep, sub, 0.0))
    t = jnp.concatenate(masked_parts, axis=0)

    rows64 = lax.broadcasted_iota(jnp.int32, (NUM_EXPERTS, bt), 0)
    idxs, wgts = [], []
    for _ in range(TOP_K):
        m = jnp.max(t, axis=0, keepdims=True)
        idx = jnp.min(jnp.where(t == m, rows64, NUM_EXPERTS), axis=0, keepdims=True)
        sel = rows64 == idx
        w = jnp.max(jnp.where(sel, scores, -jnp.inf), axis=0, keepdims=True)
        t = jnp.where(sel, -jnp.inf, t)
        idxs.append(idx)
        wgts.append(w)
    idx8 = jnp.concatenate(idxs, axis=0)
    w8 = jnp.concatenate(wgts, axis=0)
    w8 = w8 / (jnp.sum(w8, axis=0, keepdims=True) + 1e-20) * ROUTED_SCALING_FACTOR

    idx_ref[...] = idx8.T
    wgt_ref[...] = w8.T


def kernel(hidden_states, weight, e_score_correction_bias):
    hidden = hidden_states.shape[-1]
    hs = hidden_states.reshape(-1, hidden)
    n_tokens = hs.shape[0]
    wt = weight.T
    bias = e_score_correction_bias.reshape(NUM_EXPERTS, 1)

    grid = (n_tokens // BT,)
    hh = hidden // 2
    idx, wgt = pl.pallas_call(
        _gate_block,
        grid=grid,
        in_specs=[
            pl.BlockSpec((BT, hh), lambda i: (i, 0)),
            pl.BlockSpec((BT, hh), lambda i: (i, 1)),
            pl.BlockSpec((hh, NUM_EXPERTS), lambda i: (0, 0)),
            pl.BlockSpec((hh, NUM_EXPERTS), lambda i: (1, 0)),
            pl.BlockSpec((NUM_EXPERTS, 1), lambda i: (0, 0)),
        ],
        out_specs=[
            pl.BlockSpec((BT, TOP_K), lambda i: (i, 0)),
            pl.BlockSpec((BT, TOP_K), lambda i: (i, 0)),
        ],
        out_shape=[
            jax.ShapeDtypeStruct((n_tokens, TOP_K), jnp.int32),
            jax.ShapeDtypeStruct((n_tokens, TOP_K), jnp.float32),
        ],
        compiler_params=pltpu.CompilerParams(
            dimension_semantics=("arbitrary",),
            vmem_limit_bytes=128 * 1024 * 1024,
        ),
    )(hs, hs, wt, wt, bias)
    return idx, wgt


# speedup vs baseline: 2.8940x; 1.0018x over previous
import jax
import jax.numpy as jnp
from jax import lax
from jax.experimental import pallas as pl
from jax.experimental.pallas import tpu as pltpu

NUM_EXPERTS = 64
TOP_K = 8
N_GROUP = 8
GROUP_SIZE = NUM_EXPERTS // N_GROUP
TOPK_GROUP = 4
ROUTED_SCALING_FACTOR = 2.5

BT = 1024


def _gate_block(h_ref, wt_ref, bias_ref, idx_ref, wgt_ref):
    logits = jnp.dot(h_ref[...], wt_ref[...], preferred_element_type=jnp.float32)
    lt = logits.T
    scores = jax.nn.sigmoid(lt)
    sfc = scores + bias_ref[...]

    bt = lt.shape[1]
    rows8 = lax.broadcasted_iota(jnp.int32, (N_GROUP, bt), 0)

    gs_list = []
    for g in range(N_GROUP):
        sub = sfc[g * GROUP_SIZE:(g + 1) * GROUP_SIZE, :]
        m1 = jnp.max(sub, axis=0, keepdims=True)
        first = jnp.min(jnp.where(sub == m1, rows8, N_GROUP), axis=0, keepdims=True)
        m2 = jnp.max(jnp.where(rows8 == first, -jnp.inf, sub), axis=0, keepdims=True)
        gs_list.append(m1 + m2)
    gs = jnp.concatenate(gs_list, axis=0)

    masked_parts = []
    for g in range(N_GROUP):
        row = gs[g:g + 1, :]
        better = (gs > row) | ((gs == row) & (rows8 < g))
        rank = jnp.sum(better.astype(jnp.int32), axis=0, keepdims=True)
        keep = rank < TOPK_GROUP
        sub = sfc[g * GROUP_SIZE:(g + 1) * GROUP_SIZE, :]
        masked_parts.append(jnp.where(keep, sub, 0.0))
    t = jnp.concatenate(masked_parts, axis=0)

    rows64 = lax.broadcasted_iota(jnp.int32, (NUM_EXPERTS, bt), 0)
    idxs, wgts = [], []
    for _ in range(TOP_K):
        m = jnp.max(t, axis=0, keepdims=True)
        idx = jnp.min(jnp.where(t == m, rows64, NUM_EXPERTS), axis=0, keepdims=True)
        sel = rows64 == idx
        w = jnp.max(jnp.where(sel, scores, -jnp.inf), axis=0, keepdims=True)
        t = jnp.where(sel, -jnp.inf, t)
        idxs.append(idx)
        wgts.append(w)
    idx8 = jnp.concatenate(idxs, axis=0)
    w8 = jnp.concatenate(wgts, axis=0)
    w8 = w8 / (jnp.sum(w8, axis=0, keepdims=True) + 1e-20) * ROUTED_SCALING_FACTOR

    idx_ref[...] = idx8.T
    wgt_ref[...] = w8.T


def kernel(hidden_states, weight, e_score_correction_bias):
    hidden = hidden_states.shape[-1]
    hs = hidden_states.reshape(-1, hidden)
    n_tokens = hs.shape[0]
    wt = weight.T
    bias = e_score_correction_bias.reshape(NUM_EXPERTS, 1)

    grid = (n_tokens // BT,)
    idx, wgt = pl.pallas_call(
        _gate_block,
        grid=grid,
        in_specs=[
            pl.BlockSpec((BT, hidden), lambda i: (i, 0)),
            pl.BlockSpec((hidden, NUM_EXPERTS), lambda i: (0, 0)),
            pl.BlockSpec((NUM_EXPERTS, 1), lambda i: (0, 0)),
        ],
        out_specs=[
            pl.BlockSpec((BT, TOP_K), lambda i: (i, 0)),
            pl.BlockSpec((BT, TOP_K), lambda i: (i, 0)),
        ],
        out_shape=[
            jax.ShapeDtypeStruct((n_tokens, TOP_K), jnp.int32),
            jax.ShapeDtypeStruct((n_tokens, TOP_K), jnp.float32),
        ],
        compiler_params=pltpu.CompilerParams(
            dimension_semantics=("arbitrary",),
            vmem_limit_bytes=128 * 1024 * 1024,
        ),
    )(hs, wt, bias)
    return idx, wgt
